# Initial kernel scaffold; baseline (speedup 1.0000x reference)
import jax
import jax.numpy as jnp
from jax.experimental import pallas as pl


def kernel(x, Wf, bf, Wx1, bx1, Wx2, bx2, Wc, bc, Wd1, bd1, Wd2, bd2, embeds):
    raise NotImplementedError("write your pallas kernel here")


# speedup vs baseline: 5.6209x; 5.6209x over previous
import functools

import jax
import jax.numpy as jnp
from jax import lax
from jax.experimental import pallas as pl
from jax.experimental.pallas import tpu as pltpu

_IN_CH = 512
_CH = 512
_EMBED_DIM = 65920
_NUM_EMB = 1024
_NUM_CB = 8
_DIM = _EMBED_DIM // _NUM_CB
_DEC_IN = 128
_DEC_H = 256
_DATA_Y = 128

_NC = 5
_CD = _DIM // _NC


def _dot_t(a, w, precision=lax.Precision.HIGHEST):
    return lax.dot_general(
        a, w, (((1,), (1,)), ((), ())),
        preferred_element_type=jnp.float32, precision=precision)


def _vq_kernel(x_r, wf_r, bf_r, wx1_r, bx1_r, wx2_r, bx2_r,
               wd1_r, bd1_r, wd2_r, bd2_r, wc_r, bc_r, emb_r,
               dec_o, diffs_o, perp_o, pre_ref, acc_ref, zsq_ref):
    k = pl.program_id(0)
    c = pl.program_id(1)

    @pl.when((k == 0) & (c == 0))
    def _init():
        xv = x_r[...]
        pre = jnp.maximum(_dot_t(xv, wf_r[...]) + bf_r[...], 0.0)
        pre_ref[...] = pre
        e1 = jnp.maximum(_dot_t(xv, wx1_r[...]) + bx1_r[...], 0.0)
        e2 = _dot_t(e1, wx2_r[...]) + bx2_r[...]
        d1 = jnp.maximum(_dot_t(e2, wd1_r[...]) + bd1_r[...], 0.0)
        dec_o[...] = _dot_t(d1, wd2_r[...]) + bd2_r[...]
        lg = jnp.log(jnp.float32(1.0) + jnp.float32(1e-10))
        perp_o[...] = jnp.full((1, 1), jnp.exp(-jnp.float32(_NUM_CB) * lg),
                               jnp.float32)
        diffs_o[...] = jnp.zeros((1, 1), jnp.float32)

    wc = wc_r[0, 0]
    emb = emb_r[0, 0]
    bc = bc_r[0, 0]

    z = _dot_t(pre_ref[...], wc, precision=lax.Precision.DEFAULT) + bc
    cross = lax.dot_general(
        z, emb, (((1,), (0,)), ((), ())),
        preferred_element_type=jnp.float32,
        precision=lax.Precision.DEFAULT)
    esq = jnp.sum(emb * emb, axis=0, keepdims=True)
    contrib = esq - 2.0 * cross
    zsq = jnp.sum(z * z).reshape(1, 1)

    @pl.when(c == 0)
    def _first():
        acc_ref[...] = contrib
        zsq_ref[...] = zsq

    @pl.when(c > 0)
    def _rest():
        acc_ref[...] = acc_ref[...] + contrib
        zsq_ref[...] = zsq_ref[...] + zsq

    @pl.when(c == _NC - 1)
    def _finish_cb():
        dist = acc_ref[...] + zsq_ref[...]
        m = jnp.min(dist)
        diffs_o[...] = diffs_o[...] + jnp.full((1, 1), m / _DIM, jnp.float32)


@functools.partial(jax.jit, static_argnums=())
def kernel(x, Wf, bf, Wx1, bx1, Wx2, bx2, Wc, bc, Wd1, bd1, Wd2, bd2, embeds):
    wc4 = Wc.reshape(_NUM_CB, _NC, _CD, _CH)
    bc4 = bc.reshape(_NUM_CB, _NC, 1, _CD)
    emb4 = embeds.reshape(_NUM_CB, _NC, _CD, _NUM_EMB)

    const2 = lambda shape: pl.BlockSpec(shape, lambda k, c: (0, 0))
    dec, diffs, perp = pl.pallas_call(
        _vq_kernel,
        grid=(_NUM_CB, _NC),
        in_specs=[
            const2((1, _IN_CH)),
            const2((_CH, _IN_CH)),
            const2((1, _CH)),
            const2((_CH, _IN_CH)),
            const2((1, _CH)),
            const2((_DEC_IN, _CH)),
            const2((1, _DEC_IN)),
            const2((_DEC_H, _DEC_IN)),
            const2((1, _DEC_H)),
            const2((_DATA_Y, _DEC_H)),
            const2((1, _DATA_Y)),
            pl.BlockSpec((1, 1, _CD, _CH), lambda k, c: (k, c, 0, 0)),
            pl.BlockSpec((1, 1, 1, _CD), lambda k, c: (k, c, 0, 0)),
            pl.BlockSpec((1, 1, _CD, _NUM_EMB), lambda k, c: (k, c, 0, 0)),
        ],
        out_specs=[
            pl.BlockSpec((1, _DATA_Y), lambda k, c: (0, 0)),
            pl.BlockSpec((1, 1), lambda k, c: (0, 0)),
            pl.BlockSpec((1, 1), lambda k, c: (0, 0)),
        ],
        out_shape=[
            jax.ShapeDtypeStruct((1, _DATA_Y), jnp.float32),
            jax.ShapeDtypeStruct((1, 1), jnp.float32),
            jax.ShapeDtypeStruct((1, 1), jnp.float32),
        ],
        scratch_shapes=[
            pltpu.VMEM((1, _CH), jnp.float32),
            pltpu.VMEM((1, _NUM_EMB), jnp.float32),
            pltpu.VMEM((1, 1), jnp.float32),
        ],
    )(x, Wf, bf.reshape(1, _CH), Wx1, bx1.reshape(1, _CH),
      Wx2, bx2.reshape(1, _DEC_IN), Wd1, bd1.reshape(1, _DEC_H),
      Wd2, bd2.reshape(1, _DATA_Y), wc4, bc4, emb4)

    return dec, diffs.reshape(()), perp.reshape(())


# speedup vs baseline: 5.6290x; 1.0014x over previous
import functools

import jax
import jax.numpy as jnp
from jax import lax
from jax.experimental import pallas as pl
from jax.experimental.pallas import tpu as pltpu

_IN_CH = 512
_CH = 512
_EMBED_DIM = 65920
_NUM_EMB = 1024
_NUM_CB = 8
_DIM = _EMBED_DIM // _NUM_CB
_DEC_IN = 128
_DEC_H = 256
_DATA_Y = 128

_NC = 5
_CD = _DIM // _NC


def _dot_t(a, w, precision=lax.Precision.HIGHEST):
    return lax.dot_general(
        a, w, (((1,), (1,)), ((), ())),
        preferred_element_type=jnp.float32, precision=precision)


def _vq_kernel(x_r, wf_r, bf_r, wx1_r, bx1_r, wx2_r, bx2_r,
               wd1_r, bd1_r, wd2_r, bd2_r, wc_r, bc_r, emb_r,
               dec_o, diffs_o, perp_o, pre_ref, acc_ref, zsq_ref):
    k = pl.program_id(0)
    c = pl.program_id(1)

    @pl.when((k == 0) & (c == 0))
    def _init():
        xv = x_r[...]
        pre = jnp.maximum(_dot_t(xv, wf_r[...]) + bf_r[...], 0.0)
        pre_ref[...] = pre
        e1 = jnp.maximum(_dot_t(xv, wx1_r[...]) + bx1_r[...], 0.0)
        e2 = _dot_t(e1, wx2_r[...]) + bx2_r[...]
        d1 = jnp.maximum(_dot_t(e2, wd1_r[...]) + bd1_r[...], 0.0)
        dec_o[...] = _dot_t(d1, wd2_r[...]) + bd2_r[...]
        lg = jnp.log(jnp.float32(1.0) + jnp.float32(1e-10))
        perp_o[...] = jnp.full((1, 1), jnp.exp(-jnp.float32(_NUM_CB) * lg),
                               jnp.float32)
        diffs_o[...] = jnp.zeros((1, 1), jnp.float32)

    wc = wc_r[0, 0]
    emb = emb_r[0, 0]
    bc = bc_r[0, 0]

    z = lax.dot_general(
        pre_ref[...].astype(jnp.bfloat16), wc.astype(jnp.bfloat16),
        (((1,), (1,)), ((), ())),
        preferred_element_type=jnp.float32) + bc
    emb_bf = emb.astype(jnp.bfloat16)
    cross = lax.dot_general(
        z.astype(jnp.bfloat16), emb_bf, (((1,), (0,)), ((), ())),
        preferred_element_type=jnp.float32)
    esq = jnp.sum(emb * emb, axis=0, keepdims=True)
    contrib = esq - 2.0 * cross
    zsq = jnp.sum(z * z).reshape(1, 1)

    @pl.when(c == 0)
    def _first():
        acc_ref[...] = contrib
        zsq_ref[...] = zsq

    @pl.when(c > 0)
    def _rest():
        acc_ref[...] = acc_ref[...] + contrib
        zsq_ref[...] = zsq_ref[...] + zsq

    @pl.when(c == _NC - 1)
    def _finish_cb():
        dist = acc_ref[...] + zsq_ref[...]
        m = jnp.min(dist)
        diffs_o[...] = diffs_o[...] + jnp.full((1, 1), m / _DIM, jnp.float32)


@functools.partial(jax.jit, static_argnums=())
def kernel(x, Wf, bf, Wx1, bx1, Wx2, bx2, Wc, bc, Wd1, bd1, Wd2, bd2, embeds):
    wc4 = Wc.reshape(_NUM_CB, _NC, _CD, _CH)
    bc4 = bc.reshape(_NUM_CB, _NC, 1, _CD)
    emb4 = embeds.reshape(_NUM_CB, _NC, _CD, _NUM_EMB)

    const2 = lambda shape: pl.BlockSpec(shape, lambda k, c: (0, 0))
    dec, diffs, perp = pl.pallas_call(
        _vq_kernel,
        grid=(_NUM_CB, _NC),
        in_specs=[
            const2((1, _IN_CH)),
            const2((_CH, _IN_CH)),
            const2((1, _CH)),
            const2((_CH, _IN_CH)),
            const2((1, _CH)),
            const2((_DEC_IN, _CH)),
            const2((1, _DEC_IN)),
            const2((_DEC_H, _DEC_IN)),
            const2((1, _DEC_H)),
            const2((_DATA_Y, _DEC_H)),
            const2((1, _DATA_Y)),
            pl.BlockSpec((1, 1, _CD, _CH), lambda k, c: (k, c, 0, 0)),
            pl.BlockSpec((1, 1, 1, _CD), lambda k, c: (k, c, 0, 0)),
            pl.BlockSpec((1, 1, _CD, _NUM_EMB), lambda k, c: (k, c, 0, 0)),
        ],
        out_specs=[
            pl.BlockSpec((1, _DATA_Y), lambda k, c: (0, 0)),
            pl.BlockSpec((1, 1), lambda k, c: (0, 0)),
            pl.BlockSpec((1, 1), lambda k, c: (0, 0)),
        ],
        out_shape=[
            jax.ShapeDtypeStruct((1, _DATA_Y), jnp.float32),
            jax.ShapeDtypeStruct((1, 1), jnp.float32),
            jax.ShapeDtypeStruct((1, 1), jnp.float32),
        ],
        scratch_shapes=[
            pltpu.VMEM((1, _CH), jnp.float32),
            pltpu.VMEM((1, _NUM_EMB), jnp.float32),
            pltpu.VMEM((1, 1), jnp.float32),
        ],
    )(x, Wf, bf.reshape(1, _CH), Wx1, bx1.reshape(1, _CH),
      Wx2, bx2.reshape(1, _DEC_IN), Wd1, bd1.reshape(1, _DEC_H),
      Wd2, bd2.reshape(1, _DATA_Y), wc4, bc4, emb4)

    return dec, diffs.reshape(()), perp.reshape(())
